# baseline (device time: 13158 ns/iter reference)
import jax
import jax.numpy as jnp
from jax import lax
from jax.experimental import pallas as pl
from jax.experimental.pallas import tpu as pltpu

N_DEV = 4
N_LAYERS = 3
B = 128
D = 128
H = 256
RB = B // N_DEV

WIN_OFF = 128
WOUT_OFF = 128 + N_LAYERS * H

SEND_ORDER = (2, 1, 3)
RECV_ORDER = (1, 3, 2)

BF16 = jnp.bfloat16
F32 = jnp.float32


def kernel(x, Win0, Wout0, Win1, Wout1, Win2, Wout2):
    TOT = D + N_LAYERS * H + N_LAYERS * D
    def place(a, col):
        r, c = a.shape
        return jnp.pad(a, ((0, H - r), (col, TOT - col - c)))
    packed = (place(x, 0)
              + place(Win0, WIN_OFF) + place(Win1, WIN_OFF + H)
              + place(Win2, WIN_OFF + 2 * H)
              + place(Wout0, WOUT_OFF) + place(Wout1, WOUT_OFF + D)
              + place(Wout2, WOUT_OFF + 2 * D)).astype(BF16)

    def body(p_ref, out_ref, partial_ref, comm_ref, rs_ref,
             send_sems, recv_sems):
        my = lax.axis_index("i")

        barrier_sem = pltpu.get_barrier_semaphore()
        for d in range(1, N_DEV):
            peer = lax.rem(my + d, N_DEV)
            pl.semaphore_signal(
                barrier_sem, inc=1,
                device_id=(peer,), device_id_type=pl.DeviceIdType.MESH,
            )

        def layer(xv, r):
            win = p_ref[0:B, WIN_OFF + r * H:WIN_OFF + (r + 1) * H]
            wout = p_ref[:, WOUT_OFF + r * D:WOUT_OFF + (r + 1) * D]
            h = jnp.maximum(
                jnp.dot(xv, win, preferred_element_type=F32), 0.0
            ).astype(BF16)
            return jnp.dot(h, wout, preferred_element_type=F32)

        pending_sends = []
        xv = p_ref[0:B, 0:D]

        for r in range(N_LAYERS - 1):
            partial = layer(xv, r).astype(BF16)
            partial_ref[r] = partial
            if r == 0:
                pl.semaphore_wait(barrier_sem, N_DEV - 1)

            rdmas = {}
            for d in SEND_ORDER:
                peer = lax.rem(my + d, N_DEV)
                rdma = pltpu.make_async_remote_copy(
                    src_ref=partial_ref.at[r],
                    dst_ref=comm_ref.at[r, d - 1],
                    send_sem=send_sems.at[r, d - 1],
                    recv_sem=recv_sems.at[r, d - 1],
                    device_id=(peer,),
                    device_id_type=pl.DeviceIdType.MESH,
                )
                rdma.start()
                rdmas[d] = rdma
            acc = partial
            for d in RECV_ORDER:
                rdmas[d].wait_recv()
                acc = acc + comm_ref[r, d - 1]
            pending_sends.extend(rdmas.values())
            xv = acc

        r = N_LAYERS - 1
        partial_ref[r] = layer(xv, r).astype(BF16)

        rdmas = {}
        for d in SEND_ORDER:
            peer = lax.rem(my + d, N_DEV)
            rdma = pltpu.make_async_remote_copy(
                src_ref=partial_ref.at[r, pl.ds(peer * RB, RB), :],
                dst_ref=rs_ref.at[d - 1],
                send_sem=send_sems.at[r, d - 1],
                recv_sem=recv_sems.at[r, d - 1],
                device_id=(peer,),
                device_id_type=pl.DeviceIdType.MESH,
            )
            rdma.start()
            rdmas[d] = rdma
        acc = partial_ref[r, pl.ds(my * RB, RB), :].astype(F32)
        for d in RECV_ORDER:
            rdmas[d].wait_recv()
            acc = acc + rs_ref[d - 1].astype(F32)
        pending_sends.extend(rdmas.values())

        out_ref[:, :] = acc

        for rdma in pending_sends:
            rdma.wait_send()

    return pl.pallas_call(
        body,
        out_shape=jax.ShapeDtypeStruct((RB, D), F32),
        in_specs=[pl.BlockSpec(memory_space=pltpu.MemorySpace.VMEM)],
        out_specs=pl.BlockSpec(memory_space=pltpu.MemorySpace.VMEM),
        scratch_shapes=[
            pltpu.VMEM((N_LAYERS, B, D), BF16),
            pltpu.VMEM((N_LAYERS - 1, N_DEV - 1, B, D), BF16),
            pltpu.VMEM((N_DEV - 1, RB, D), BF16),
            pltpu.SemaphoreType.DMA((N_LAYERS, N_DEV - 1)),
            pltpu.SemaphoreType.DMA((N_LAYERS, N_DEV - 1)),
        ],
        compiler_params=pltpu.CompilerParams(collective_id=0),
    )(packed)


# device time: 13097 ns/iter; 1.0047x vs baseline; 1.0047x over previous
import jax
import jax.numpy as jnp
from jax import lax
from jax.experimental import pallas as pl
from jax.experimental.pallas import tpu as pltpu

N_DEV = 4
N_LAYERS = 3
B = 128
D = 128
H = 256
RB = B // N_DEV

WIN_OFF = 128

SEND_ORDER = (2, 1, 3)
RECV_ORDER = (1, 3, 2)

BF16 = jnp.bfloat16
F32 = jnp.float32


def kernel(x, Win0, Wout0, Win1, Wout1, Win2, Wout2):
    TOT_A = D + N_LAYERS * H
    TOT_B = N_LAYERS * D
    def place(a, col, tot):
        return jnp.pad(a, ((0, 0), (col, tot - col - a.shape[1])))
    packed_a = (place(x, 0, TOT_A)
                + place(Win0, WIN_OFF, TOT_A)
                + place(Win1, WIN_OFF + H, TOT_A)
                + place(Win2, WIN_OFF + 2 * H, TOT_A)).astype(BF16)
    packed_b = (place(Wout0, 0, TOT_B)
                + place(Wout1, D, TOT_B)
                + place(Wout2, 2 * D, TOT_B)).astype(BF16)

    def body(pa_ref, pb_ref, out_ref, partial_ref, comm_ref, rs_ref,
             send_sems, recv_sems):
        my = lax.axis_index("i")

        barrier_sem = pltpu.get_barrier_semaphore()
        for d in range(1, N_DEV):
            peer = lax.rem(my + d, N_DEV)
            pl.semaphore_signal(
                barrier_sem, inc=1,
                device_id=(peer,), device_id_type=pl.DeviceIdType.MESH,
            )

        def layer(xv, r):
            win = pa_ref[:, WIN_OFF + r * H:WIN_OFF + (r + 1) * H]
            wout = pb_ref[:, r * D:(r + 1) * D]
            h = jnp.maximum(
                jnp.dot(xv, win, preferred_element_type=F32), 0.0
            ).astype(BF16)
            return jnp.dot(h, wout, preferred_element_type=F32)

        pending_sends = []
        xv = pa_ref[:, 0:D]

        for r in range(N_LAYERS - 1):
            partial = layer(xv, r).astype(BF16)
            partial_ref[r] = partial
            if r == 0:
                pl.semaphore_wait(barrier_sem, N_DEV - 1)

            rdmas = {}
            for d in SEND_ORDER:
                peer = lax.rem(my + d, N_DEV)
                rdma = pltpu.make_async_remote_copy(
                    src_ref=partial_ref.at[r],
                    dst_ref=comm_ref.at[r, d - 1],
                    send_sem=send_sems.at[r, d - 1],
                    recv_sem=recv_sems.at[r, d - 1],
                    device_id=(peer,),
                    device_id_type=pl.DeviceIdType.MESH,
                )
                rdma.start()
                rdmas[d] = rdma
            acc = partial
            for d in RECV_ORDER:
                rdmas[d].wait_recv()
                acc = acc + comm_ref[r, d - 1]
            pending_sends.extend(rdmas.values())
            xv = acc

        r = N_LAYERS - 1
        partial_ref[r] = layer(xv, r).astype(BF16)

        rdmas = {}
        for d in SEND_ORDER:
            peer = lax.rem(my + d, N_DEV)
            rdma = pltpu.make_async_remote_copy(
                src_ref=partial_ref.at[r, pl.ds(peer * RB, RB), :],
                dst_ref=rs_ref.at[d - 1],
                send_sem=send_sems.at[r, d - 1],
                recv_sem=recv_sems.at[r, d - 1],
                device_id=(peer,),
                device_id_type=pl.DeviceIdType.MESH,
            )
            rdma.start()
            rdmas[d] = rdma
        acc = partial_ref[r, pl.ds(my * RB, RB), :].astype(F32)
        for d in RECV_ORDER:
            rdmas[d].wait_recv()
            acc = acc + rs_ref[d - 1].astype(F32)
        pending_sends.extend(rdmas.values())

        out_ref[:, :] = acc

        for rdma in pending_sends:
            rdma.wait_send()

    return pl.pallas_call(
        body,
        out_shape=jax.ShapeDtypeStruct((RB, D), F32),
        in_specs=[pl.BlockSpec(memory_space=pltpu.MemorySpace.VMEM)] * 2,
        out_specs=pl.BlockSpec(memory_space=pltpu.MemorySpace.VMEM),
        scratch_shapes=[
            pltpu.VMEM((N_LAYERS, B, D), BF16),
            pltpu.VMEM((N_LAYERS - 1, N_DEV - 1, B, D), BF16),
            pltpu.VMEM((N_DEV - 1, RB, D), BF16),
            pltpu.SemaphoreType.DMA((N_LAYERS, N_DEV - 1)),
            pltpu.SemaphoreType.DMA((N_LAYERS, N_DEV - 1)),
        ],
        compiler_params=pltpu.CompilerParams(collective_id=0),
    )(packed_a, packed_b)
